# Initial kernel scaffold; baseline (speedup 1.0000x reference)
import jax
import jax.numpy as jnp
from jax.experimental import pallas as pl


def kernel(x, edge_index, community, W_lin, b_lin, W1, b1, W2, b2):
    raise NotImplementedError("write your pallas kernel here")


# speedup vs baseline: 15.5585x; 15.5585x over previous
import functools

import jax
import jax.numpy as jnp
from jax import lax
from jax.experimental import pallas as pl
from jax.experimental.pallas import tpu as pltpu
from jax.experimental.pallas import tpu_sc as plsc

N = 10000
E = 320000
D = 128
H = 128
C = 40
NCOMM = 100

NP = 10240
WP = 48
NCORE = 2
NSUB = 16
NWORK = NCORE * NSUB
CHUNK = 128
NCH = 79
EPT = NCH * CHUNK
EPAD = NWORK * EPT
ROWS_PER_SUB = NP // NSUB
DUMP_ROW = N + 64

_f32 = jnp.float32
_i32 = jnp.int32


def _h0_body(x_ref, comm_ref, wlin_ref, blin_ref, h0_ref):
    x = x_ref[...]
    comm = comm_ref[...]
    ids = lax.broadcasted_iota(_i32, (N, NCOMM), 1)
    onehot = (comm == ids).astype(_f32)
    csum = lax.dot_general(onehot, x, (((0,), (0,)), ((), ())),
                           preferred_element_type=_f32)
    cnt = jnp.sum(onehot, axis=0)[:, None]
    cmean = csum / jnp.maximum(cnt, 1.0)
    xc = jnp.dot(onehot, cmean, preferred_element_type=_f32)
    wlin = wlin_ref[...]
    h0 = x @ wlin[0:D] + xc @ wlin[D:2 * D] + blin_ref[...]
    h0_ref[...] = jnp.maximum(h0, 0.0)


def _h0_call(x, comm2d, W_lin, blin2d):
    return pl.pallas_call(
        _h0_body,
        out_shape=jax.ShapeDtypeStruct((N, H), _f32),
    )(x, comm2d, W_lin, blin2d)


def _deg_body(dst_hbm, out_hbm, idx_v, acc_v):
    cid = lax.axis_index("c")
    sid = lax.axis_index("s")
    wid = sid * NCORE + cid
    pltpu.sync_copy(dst_hbm.at[wid], idx_v)

    def _zero(i, _):
        acc_v[pl.ds(i * 16, 16)] = jnp.zeros((16,), _f32)
        return 0
    lax.fori_loop(0, NP // 16, _zero, 0)

    ones16 = jnp.full((16,), 1.0, _f32)

    def _edges(c, _):
        def _sub(j, __):
            idx = idx_v[c, pl.ds(j * 16, 16)]
            plsc.addupdate_scatter(acc_v, [idx], ones16)
            return 0
        lax.fori_loop(0, 8, _sub, 0)
        return 0
    lax.fori_loop(0, NCH, _edges, 0)
    pltpu.sync_copy(acc_v, out_hbm.at[wid])


_deg_call = functools.partial(
    pl.kernel,
    out_type=jax.ShapeDtypeStruct((NWORK, NP), _f32),
    mesh=plsc.VectorSubcoreMesh(core_axis_name="c", subcore_axis_name="s"),
    compiler_params=pltpu.CompilerParams(needs_layout_passes=False),
    scratch_types=[
        pltpu.VMEM((NCH, CHUNK), _i32),
        pltpu.VMEM((NP,), _f32),
    ],
)(_deg_body)


def _prep_body(degp_ref, h0_ref, dinv_ref, g_ref):
    deg = jnp.sum(degp_ref[...], axis=0) + 1.0
    dinv = lax.rsqrt(deg)[:, None]
    dinv_ref[...] = dinv
    g_ref[0:N, :] = h0_ref[...] * dinv[0:N]
    g_ref[N:NP, :] = jnp.zeros((NP - N, D), _f32)


def _prep_call(degp, h0):
    return pl.pallas_call(
        _prep_body,
        out_shape=(
            jax.ShapeDtypeStruct((NP, 1), _f32),
            jax.ShapeDtypeStruct((NP, D), _f32),
        ),
    )(degp, h0)


def _make_agg(width):
    def _body(g_hbm, src_hbm, dst_hbm, out_hbm, src_v, dst_v, buf_v, acc_sh, sem):
        cid = lax.axis_index("c")
        sid = lax.axis_index("s")
        wid = sid * NCORE + cid
        pltpu.sync_copy(src_hbm.at[wid], src_v)
        pltpu.sync_copy(dst_hbm.at[wid], dst_v)

        def _zrow(i, _):
            def _zf(f, __):
                buf_v[i, pl.ds(f * 16, 16)] = jnp.zeros((16,), _f32)
                return 0
            lax.fori_loop(0, width // 16, _zf, 0)
            return 0
        lax.fori_loop(0, CHUNK, _zrow, 0)
        for k in range(ROWS_PER_SUB // CHUNK):
            pltpu.sync_copy(buf_v, acc_sh.at[pl.ds(sid * ROWS_PER_SUB + k * CHUNK, CHUNK), :])
        plsc.subcore_barrier()

        def _edges(c, _):
            pltpu.async_copy(g_hbm.at[src_v.at[c]], buf_v, sem).wait()
            pltpu.sync_copy(buf_v, acc_sh.at[dst_v.at[c]], add=True)
            return 0
        lax.fori_loop(0, NCH, _edges, 0)
        plsc.subcore_barrier()
        pltpu.sync_copy(acc_sh.at[pl.ds(sid * ROWS_PER_SUB, ROWS_PER_SUB), :],
                        out_hbm.at[cid, pl.ds(sid * ROWS_PER_SUB, ROWS_PER_SUB), :])

    return functools.partial(
        pl.kernel,
        out_type=jax.ShapeDtypeStruct((NCORE, NP, width), _f32),
        mesh=plsc.VectorSubcoreMesh(core_axis_name="c", subcore_axis_name="s"),
        compiler_params=pltpu.CompilerParams(
            needs_layout_passes=False,
            use_tc_tiling_on_sc=False if width % 128 else None,
        ),
        scratch_types=[
            pltpu.VMEM((NCH, CHUNK), _i32),
            pltpu.VMEM((NCH, CHUNK), _i32),
            pltpu.VMEM((CHUNK, width), _f32),
            pltpu.VMEM_SHARED((NP, width), _f32),
            pltpu.SemaphoreType.DMA,
        ],
    )(_body)


_agg_d = _make_agg(D)
_agg_w = _make_agg(WP)


def _mid_body(aggp_ref, g_ref, dinv_ref, w1_ref, b1_ref, w2_ref, q_ref):
    s = aggp_ref[0] + aggp_ref[1]
    dinv = dinv_ref[...]
    agg1 = dinv * (s + g_ref[...])
    h1 = jnp.maximum(agg1 @ w1_ref[...] + b1_ref[...], 0.0)
    q_ref[...] = dinv * (h1 @ w2_ref[...])


def _mid_call(aggp, g, dinv, W1, b1_2d, W2p):
    return pl.pallas_call(
        _mid_body,
        out_shape=jax.ShapeDtypeStruct((NP, WP), _f32),
    )(aggp, g, dinv, W1, b1_2d, W2p)


def _out_body(agg2p_ref, q_ref, dinv_ref, b2_ref, out_ref):
    s2 = agg2p_ref[0] + agg2p_ref[1]
    o = dinv_ref[...] * (s2 + q_ref[...])
    out_ref[...] = o[0:N, 0:C] + b2_ref[...]


def _out_call(agg2p, q, dinv, b2_2d):
    return pl.pallas_call(
        _out_body,
        out_shape=jax.ShapeDtypeStruct((N, C), _f32),
    )(agg2p, q, dinv, b2_2d)


def kernel(x, edge_index, community, W_lin, b_lin, W1, b1, W2, b2):
    src = edge_index[0]
    dst = edge_index[1]
    pad = EPAD - E
    src3 = jnp.concatenate([src, jnp.zeros((pad,), _i32)]).reshape(NWORK, NCH, CHUNK)
    dst3 = jnp.concatenate([dst, jnp.full((pad,), DUMP_ROW, _i32)]).reshape(NWORK, NCH, CHUNK)
    W2p = jnp.pad(W2, ((0, 0), (0, WP - C)))

    h0 = _h0_call(x, community.reshape(N, 1), W_lin, b_lin.reshape(1, H))
    degp = _deg_call(dst3)
    dinv, g = _prep_call(degp, h0)
    aggp = _agg_d(g, src3, dst3)
    q = _mid_call(aggp, g, dinv, W1, b1.reshape(1, H), W2p)
    agg2p = _agg_w(q, src3, dst3)
    return _out_call(agg2p, q, dinv, b2.reshape(1, C))
